# baseline (speedup 1.0000x reference)
import functools

import jax
import jax.numpy as jnp
from jax import lax
from jax.experimental import pallas as pl
from jax.experimental.pallas import tpu as pltpu
from jax.experimental.pallas import tpu_sc as plsc

_N_ROWS = 100000
_D = 128
_N_IDX = 425984
_NW = 32
_BPW = _N_IDX // _NW
_PR = 3128
_PR_LAST = _N_ROWS - 31 * _PR
_C = 208
_NCHUNK = _BPW // _C

_mesh = plsc.VectorSubcoreMesh(core_axis_name="c", subcore_axis_name="s")


@functools.partial(
    pl.kernel,
    mesh=_mesh,
    out_type=(
        jax.ShapeDtypeStruct((_N_ROWS, _D), jnp.float32),
        jax.ShapeDtypeStruct((_N_IDX,), jnp.int32),
        jax.ShapeDtypeStruct((_N_IDX, _D), jnp.float32),
    ),
    scratch_types=[
        pltpu.VMEM((_BPW,), jnp.int32),
        pltpu.VMEM((4, _C, _D), jnp.float32),
        pltpu.SemaphoreType.DMA,
        pltpu.SemaphoreType.DMA,
        pltpu.SemaphoreType.DMA,
        pltpu.SemaphoreType.DMA,
    ],
)
def _gather_rows(table_hbm, idx_hbm, inp_out_hbm, idx_out_hbm, out_hbm,
                 idx_v, rows_v, sem_in, sem_out, sem_p0, sem_p1):
    wid = lax.axis_index("s") * 2 + lax.axis_index("c")
    base = wid * _BPW
    prow = wid * _PR

    def pass_inp_desc(nrows):
        return pltpu.make_async_copy(
            table_hbm.at[pl.ds(prow, nrows)],
            inp_out_hbm.at[pl.ds(prow, nrows)],
            sem_p0,
        )

    @pl.when(wid < _NW - 1)
    def _():
        pass_inp_desc(_PR).start()

    @pl.when(wid == _NW - 1)
    def _():
        pass_inp_desc(_PR_LAST).start()

    pass_idx = pltpu.make_async_copy(
        idx_hbm.at[pl.ds(base, _BPW)], idx_out_hbm.at[pl.ds(base, _BPW)],
        sem_p1,
    )
    pass_idx.start()

    pltpu.sync_copy(idx_hbm.at[pl.ds(base, _BPW)], idx_v)

    def gather_desc(j, buf):
        return pltpu.make_async_copy(
            table_hbm.at[idx_v.at[pl.ds(j * _C, _C)]], rows_v.at[buf], sem_in
        )

    def out_desc(j, buf):
        return pltpu.make_async_copy(
            rows_v.at[buf], out_hbm.at[pl.ds(base + j * _C, _C)], sem_out
        )

    gather_desc(0, 0).start()
    gather_desc(1, 1).start()

    def body(j, carry):
        b = j % 4

        @pl.when(j >= 2)
        def _():
            out_desc(j - 2, (j - 2) % 4).wait()

        @pl.when(j + 2 < _NCHUNK)
        def _():
            gather_desc(j + 2, (j + 2) % 4).start()

        gather_desc(j, b).wait()
        out_desc(j, b).start()
        return carry

    lax.fori_loop(0, _NCHUNK, body, 0)
    out_desc(_NCHUNK - 2, (_NCHUNK - 2) % 4).wait()
    out_desc(_NCHUNK - 1, (_NCHUNK - 1) % 4).wait()
    @pl.when(wid < _NW - 1)
    def _():
        pass_inp_desc(_PR).wait()

    @pl.when(wid == _NW - 1)
    def _():
        pass_inp_desc(_PR_LAST).wait()

    pass_idx.wait()


def kernel(input, index, _):
    inp_out, idx_out, out = _gather_rows(input, index)
    return (inp_out, idx_out, out)


# speedup vs baseline: 8.0556x; 8.0556x over previous
import functools

import jax
import jax.numpy as jnp
from jax import lax
from jax.experimental import pallas as pl
from jax.experimental.pallas import tpu as pltpu
from jax.experimental.pallas import tpu_sc as plsc

_N_ROWS = 100000
_D = 128
_N_IDX = 425984
_NW = 32
_BPW = _N_IDX // _NW
_C = 208
_NCHUNK = _BPW // _C

_mesh = plsc.VectorSubcoreMesh(core_axis_name="c", subcore_axis_name="s")


@functools.partial(
    pl.kernel,
    mesh=_mesh,
    out_type=(
        jax.ShapeDtypeStruct((_N_IDX,), jnp.int32),
        jax.ShapeDtypeStruct((_N_IDX, _D), jnp.float32),
    ),
    scratch_types=[
        pltpu.VMEM((_BPW,), jnp.int32),
        pltpu.VMEM((4, _C, _D), jnp.float32),
        pltpu.SemaphoreType.DMA,
        pltpu.SemaphoreType.DMA,
    ],
)
def _gather_rows(table_hbm, idx_hbm, idx_out_hbm, out_hbm,
                 idx_v, rows_v, sem_in, sem_out):
    wid = lax.axis_index("s") * 2 + lax.axis_index("c")
    base = wid * _BPW
    pltpu.sync_copy(idx_hbm.at[pl.ds(base, _BPW)], idx_v)

    def gather_desc(j, buf):
        return pltpu.make_async_copy(
            table_hbm.at[idx_v.at[pl.ds(j * _C, _C)]], rows_v.at[buf], sem_in
        )

    def out_desc(j, buf):
        return pltpu.make_async_copy(
            rows_v.at[buf], out_hbm.at[pl.ds(base + j * _C, _C)], sem_out
        )

    gather_desc(0, 0).start()
    gather_desc(1, 1).start()

    def body(j, carry):
        b = j % 4

        @pl.when(j >= 2)
        def _():
            out_desc(j - 2, (j - 2) % 4).wait()

        @pl.when(j + 2 < _NCHUNK)
        def _():
            gather_desc(j + 2, (j + 2) % 4).start()

        gather_desc(j, b).wait()
        out_desc(j, b).start()
        return carry

    lax.fori_loop(0, _NCHUNK, body, 0)

    pltpu.sync_copy(idx_v, idx_out_hbm.at[pl.ds(base, _BPW)])

    out_desc(_NCHUNK - 2, (_NCHUNK - 2) % 4).wait()
    out_desc(_NCHUNK - 1, (_NCHUNK - 1) % 4).wait()


_COPY_ROWS = 800


def _copy_body(x_ref, o_ref):
    o_ref[...] = x_ref[...]


_tc_copy = pl.pallas_call(
    _copy_body,
    grid=(_N_ROWS // _COPY_ROWS,),
    in_specs=[pl.BlockSpec((_COPY_ROWS, _D), lambda i: (i, 0))],
    out_specs=pl.BlockSpec((_COPY_ROWS, _D), lambda i: (i, 0)),
    out_shape=jax.ShapeDtypeStruct((_N_ROWS, _D), jnp.float32),
)


def kernel(input, index, _):
    idx_out, out = _gather_rows(input, index)
    inp_out = _tc_copy(input)
    return (inp_out, idx_out, out)
